# baseline (speedup 1.0000x reference)
import jax
import jax.numpy as jnp
from jax import lax
from jax.experimental import pallas as pl
from jax.experimental.pallas import tpu as pltpu
from jax.experimental.pallas import tpu_sc as plsc

D = 64
L = 16
NC, NS = 2, 16
NW = NC * NS
CHUNK = 64
TRW = 512
COFF = 0.001

GCOLS = (0, 1, 2, 3, 4, 6, 7)
NG = len(GCOLS)


def _sc_body(x_hbm, pairs_hbm, out_hbm, *refs):
    rows_v = refs[0]
    out_v = refs[1:4]
    tb = refs[4:7]
    pairs_v = refs[7]
    idx_v = refs[8:15]
    par_v = refs[15:22]
    sem = refs[22]
    B = pairs_hbm.shape[0] // 9
    pw = B // NW
    nchunk = pw // CHUNK
    ngroup = CHUNK // L

    wid = lax.axis_index("s") * NC + lax.axis_index("c")
    base = wid * pw

    lanes = lax.iota(jnp.int32, L)
    lanes16 = lanes * L

    pltpu.sync_copy(pairs_hbm.at[pl.ds(base * 9, pw * 9)], pairs_v)

    def deint_body(g, carry0):
        gbase = (lanes + g * L) * 9
        for slot, col in enumerate(GCOLS):
            raw = plsc.load_gather(pairs_v, [gbase + col])
            s = lax.bitwise_and(raw, 2 * TRW - 1)
            half = lax.shift_right_logical(s, 9)
            row = (lax.shift_right_logical(raw, 10) * TRW
                   + lax.bitwise_and(s, TRW - 1))
            idx_v[slot][pl.ds(g * L, L)] = row
            par_v[slot][pl.ds(g * L, L)] = half * D
        return carry0

    lax.fori_loop(0, pw // L, deint_body, 0)

    def chunk_body(c, carry):
        copies = []
        for slot in range(NG):
            copies.append(pltpu.async_copy(
                x_hbm.at[idx_v[slot].at[pl.ds(c * CHUNK, CHUNK)]],
                rows_v.at[pl.ds(slot * CHUNK, CHUNK)],
                sem))
        for cp in copies:
            cp.wait()

        def group_body(g, carry2):
            p0 = g * L
            goff = c * CHUNK + p0
            pvec = [par_v[s][pl.ds(goff, L)] for s in range(NG)]
            for j in range(L):
                row = p0 + j
                po = [pvec[s][j] for s in range(NG)]
                u = [rows_v[0 * CHUNK + row, pl.ds(po[0] + k * L, L)] for k in range(D // L)]
                ip = [rows_v[1 * CHUNK + row, pl.ds(po[1] + k * L, L)] for k in range(D // L)]
                inn = [rows_v[2 * CHUNK + row, pl.ds(po[2] + k * L, L)] for k in range(D // L)]
                eip = [rows_v[3 * CHUNK + row, pl.ds(po[3] + k * L, L)] for k in range(D // L)]
                ein = [rows_v[4 * CHUNK + row, pl.ds(po[4] + k * L, L)] for k in range(D // L)]
                eup = [rows_v[5 * CHUNK + row, pl.ds(po[5] + k * L, L)] for k in range(D // L)]
                eun = [rows_v[6 * CHUNK + row, pl.ds(po[6] + k * L, L)] for k in range(D // L)]
                vcf = vi = vu = None
                for k in range(D // L):
                    tcf = u[k] * (ip[k] - inn[k])
                    a = ip[k] - eip[k]
                    b = ip[k] - ein[k]
                    ti = a * a - b * b
                    a = u[k] - eup[k]
                    b = u[k] - eun[k]
                    tu = a * a - b * b
                    vcf = tcf if vcf is None else vcf + tcf
                    vi = ti if vi is None else vi + ti
                    vu = tu if vu is None else vu + tu
                sidx = lanes16 + j
                plsc.store_scatter(tb[0], [sidx], vcf)
                plsc.store_scatter(tb[1], [sidx], vi)
                plsc.store_scatter(tb[2], [sidx], vu)
            zs = []
            for t in range(3):
                acc = tb[t][pl.ds(0, L)]
                for l in range(1, L):
                    acc = acc + tb[t][pl.ds(l * L, L)]
                zs.append(acc)
            zcf, zi, zu = zs
            off = c * CHUNK + p0
            mi = plsc.load_gather(pairs_v, [(lanes + off) * 9 + 5])
            mu = plsc.load_gather(pairs_v, [(lanes + off) * 9 + 8])
            out_v[0][pl.ds(off, L)] = zcf
            out_v[1][pl.ds(off, L)] = zi * mi.astype(jnp.float32)
            out_v[2][pl.ds(off, L)] = zu * mu.astype(jnp.float32)
            return carry2

        return lax.fori_loop(0, ngroup, group_body, carry)

    lax.fori_loop(0, nchunk, chunk_body, 0)
    for r in range(3):
        pltpu.sync_copy(out_v[r], out_hbm.at[pl.ds(r * B + base, pw)])


def _sc_pairs(x, pairs_flat):
    B = pairs_flat.shape[0] // 9
    mesh = plsc.VectorSubcoreMesh(
        core_axis_name="c", subcore_axis_name="s",
        num_cores=NC, num_subcores=NS)
    kfn = pl.kernel(
        _sc_body,
        out_type=jax.ShapeDtypeStruct((3 * B,), jnp.float32),
        mesh=mesh,
        compiler_params=pltpu.CompilerParams(
            needs_layout_passes=False, use_tc_tiling_on_sc=True),
        scratch_types=(
            [pltpu.VMEM((NG * CHUNK, 2 * D), jnp.float32)]
            + [pltpu.VMEM((B // NW,), jnp.float32)] * 3
            + [pltpu.VMEM((L * L,), jnp.float32)] * 3
            + [pltpu.VMEM((B // NW * 9,), jnp.int32)]
            + [pltpu.VMEM((B // NW,), jnp.int32)] * (2 * NG)
            + [pltpu.SemaphoreType.DMA]
        ),
    )
    return kfn(x, pairs_flat)


def _tr_body(lo_ref, hi_ref, o_ref):
    o_ref[:, 0:D] = lo_ref[...].T
    o_ref[:, D:2 * D] = hi_ref[...].T


def _transpose_table(xt):
    n = xt.shape[1]
    grid = (n + 2 * TRW - 1) // (2 * TRW)
    return pl.pallas_call(
        _tr_body,
        grid=(grid,),
        in_specs=[
            pl.BlockSpec((D, TRW), lambda i: (0, 2 * i)),
            pl.BlockSpec((D, TRW), lambda i: (0, 2 * i + 1)),
        ],
        out_specs=pl.BlockSpec((TRW, 2 * D), lambda i: (i, 0)),
        out_shape=jax.ShapeDtypeStruct((n // 2, 2 * D), jnp.float32),
    )(xt, xt)


def _loss_body(z_ref, o_ref):
    z = z_ref[...]
    ls = jnp.minimum(z, 0.0) - jnp.log1p(jnp.exp(-jnp.abs(z)))
    total = -(jnp.sum(ls[0, :])
              + COFF * (jnp.sum(ls[1, :]) + jnp.sum(ls[2, :])))
    o_ref[...] = jnp.reshape(total, (1, 1))


def kernel(x, pos_neg_pair_t):
    B = pos_neg_pair_t.shape[0]
    pairs_flat = pos_neg_pair_t.astype(jnp.int32).reshape(-1)
    xpk = _transpose_table(x.T)
    z = _sc_pairs(xpk, pairs_flat).reshape(3, B)
    loss2d = pl.pallas_call(
        _loss_body,
        out_shape=jax.ShapeDtypeStruct((1, 1), jnp.float32),
    )(z)
    return loss2d[0, 0]


# speedup vs baseline: 1.8473x; 1.8473x over previous
import jax
import jax.numpy as jnp
from jax import lax
from jax.experimental import pallas as pl
from jax.experimental.pallas import tpu as pltpu
from jax.experimental.pallas import tpu_sc as plsc

D = 64
L = 16
NC, NS = 2, 16
NW = NC * NS
CHUNK = 64
COFF = 0.001

GCOLS = (0, 1, 2, 3, 4, 6, 7)
NG = len(GCOLS)


def _sc_body(x_hbm, pairs_hbm, out_hbm, *refs):
    rows_v = refs[0]
    out_v = refs[1:4]
    tb = refs[4:7]
    pairs_v = refs[7]
    sem = refs[8]
    B = pairs_hbm.shape[0] // 9
    pw = B // NW
    nchunk = pw // CHUNK
    ngroup = CHUNK // L

    wid = lax.axis_index("s") * NC + lax.axis_index("c")
    base = wid * pw

    lanes = lax.iota(jnp.int32, L)
    lanes16 = lanes * L

    pltpu.sync_copy(pairs_hbm.at[pl.ds(base * 9, pw * 9)], pairs_v)

    def chunk_body(c, carry):
        def row_dma_body(g, carry1):
            gbase = (lanes + c * CHUNK + g * L) * 9
            for slot, col in enumerate(GCOLS):
                idxvec = plsc.load_gather(pairs_v, [gbase + col])
                for j in range(L):
                    pltpu.async_copy(
                        x_hbm.at[pl.ds(idxvec[j], 1), :],
                        rows_v.at[pl.ds(slot * CHUNK + g * L + j, 1), :],
                        sem)
            return carry1

        lax.fori_loop(0, CHUNK // L, row_dma_body, 0)
        pltpu.make_async_copy(x_hbm.at[pl.ds(0, NG * CHUNK), :], rows_v,
                              sem).wait()

        def group_body(g, carry2):
            p0 = g * L
            for j in range(L):
                row = p0 + j
                u = [rows_v[0 * CHUNK + row, pl.ds(k * L, L)] for k in range(D // L)]
                ip = [rows_v[1 * CHUNK + row, pl.ds(k * L, L)] for k in range(D // L)]
                inn = [rows_v[2 * CHUNK + row, pl.ds(k * L, L)] for k in range(D // L)]
                eip = [rows_v[3 * CHUNK + row, pl.ds(k * L, L)] for k in range(D // L)]
                ein = [rows_v[4 * CHUNK + row, pl.ds(k * L, L)] for k in range(D // L)]
                eup = [rows_v[5 * CHUNK + row, pl.ds(k * L, L)] for k in range(D // L)]
                eun = [rows_v[6 * CHUNK + row, pl.ds(k * L, L)] for k in range(D // L)]
                vcf = vi = vu = None
                for k in range(D // L):
                    tcf = u[k] * (ip[k] - inn[k])
                    a = ip[k] - eip[k]
                    b = ip[k] - ein[k]
                    ti = a * a - b * b
                    a = u[k] - eup[k]
                    b = u[k] - eun[k]
                    tu = a * a - b * b
                    vcf = tcf if vcf is None else vcf + tcf
                    vi = ti if vi is None else vi + ti
                    vu = tu if vu is None else vu + tu
                sidx = lanes16 + j
                plsc.store_scatter(tb[0], [sidx], vcf)
                plsc.store_scatter(tb[1], [sidx], vi)
                plsc.store_scatter(tb[2], [sidx], vu)
            zs = []
            for t in range(3):
                acc = tb[t][pl.ds(0, L)]
                for l in range(1, L):
                    acc = acc + tb[t][pl.ds(l * L, L)]
                zs.append(acc)
            zcf, zi, zu = zs
            off = c * CHUNK + p0
            mi = plsc.load_gather(pairs_v, [(lanes + off) * 9 + 5])
            mu = plsc.load_gather(pairs_v, [(lanes + off) * 9 + 8])
            out_v[0][pl.ds(off, L)] = zcf
            out_v[1][pl.ds(off, L)] = zi * mi.astype(jnp.float32)
            out_v[2][pl.ds(off, L)] = zu * mu.astype(jnp.float32)
            return carry2

        return lax.fori_loop(0, ngroup, group_body, carry)

    lax.fori_loop(0, nchunk, chunk_body, 0)
    for r in range(3):
        pltpu.sync_copy(out_v[r], out_hbm.at[pl.ds(r * B + base, pw)])


def _sc_pairs(x, pairs_flat):
    B = pairs_flat.shape[0] // 9
    mesh = plsc.VectorSubcoreMesh(
        core_axis_name="c", subcore_axis_name="s",
        num_cores=NC, num_subcores=NS)
    kfn = pl.kernel(
        _sc_body,
        out_type=jax.ShapeDtypeStruct((3 * B,), jnp.float32),
        mesh=mesh,
        compiler_params=pltpu.CompilerParams(
            needs_layout_passes=False, use_tc_tiling_on_sc=True),
        scratch_types=(
            [pltpu.VMEM((NG * CHUNK, D), jnp.float32)]
            + [pltpu.VMEM((B // NW,), jnp.float32)] * 3
            + [pltpu.VMEM((L * L,), jnp.float32)] * 3
            + [pltpu.VMEM((B // NW * 9,), jnp.int32)]
            + [pltpu.SemaphoreType.DMA]
        ),
    )
    return kfn(x, pairs_flat)


def _loss_body(z_ref, o_ref):
    z = z_ref[...]
    ls = jnp.minimum(z, 0.0) - jnp.log1p(jnp.exp(-jnp.abs(z)))
    total = -(jnp.sum(ls[0, :])
              + COFF * (jnp.sum(ls[1, :]) + jnp.sum(ls[2, :])))
    o_ref[...] = jnp.reshape(total, (1, 1))


def kernel(x, pos_neg_pair_t):
    B = pos_neg_pair_t.shape[0]
    pairs_flat = pos_neg_pair_t.astype(jnp.int32).reshape(-1)
    z = _sc_pairs(x, pairs_flat).reshape(3, B)
    loss2d = pl.pallas_call(
        _loss_body,
        out_shape=jax.ShapeDtypeStruct((1, 1), jnp.float32),
    )(z)
    return loss2d[0, 0]


# speedup vs baseline: 1.8575x; 1.0055x over previous
import jax
import jax.numpy as jnp
from jax import lax
from jax.experimental import pallas as pl
from jax.experimental.pallas import tpu as pltpu
from jax.experimental.pallas import tpu_sc as plsc

D = 64
L = 16
NC, NS = 2, 16
NW = NC * NS
CHUNK = 128
COFF = 0.001

GCOLS = (0, 1, 2, 3, 4, 6, 7)
NG = len(GCOLS)


def _sc_body(x_hbm, pairs_hbm, out_hbm, *refs):
    rows_v = refs[0]
    out_v = refs[1:4]
    tb = refs[4:7]
    pairs_v = refs[7]
    sem = refs[8]
    B = pairs_hbm.shape[0] // 9
    pw = B // NW
    nchunk = pw // CHUNK
    ngroup = CHUNK // L

    wid = lax.axis_index("s") * NC + lax.axis_index("c")
    base = wid * pw

    lanes = lax.iota(jnp.int32, L)
    lanes16 = lanes * L

    pltpu.sync_copy(pairs_hbm.at[pl.ds(base * 9, pw * 9)], pairs_v)

    def chunk_body(c, carry):
        def row_dma_body(g, carry1):
            gbase = (lanes + c * CHUNK + g * L) * 9
            for slot, col in enumerate(GCOLS):
                idxvec = plsc.load_gather(pairs_v, [gbase + col])
                for j in range(L):
                    pltpu.async_copy(
                        x_hbm.at[pl.ds(idxvec[j], 1), :],
                        rows_v.at[pl.ds(slot * CHUNK + g * L + j, 1), :],
                        sem)
            return carry1

        lax.fori_loop(0, CHUNK // L, row_dma_body, 0)
        pltpu.make_async_copy(x_hbm.at[pl.ds(0, NG * CHUNK), :], rows_v,
                              sem).wait()

        def group_body(g, carry2):
            p0 = g * L
            for j in range(L):
                row = p0 + j
                u = [rows_v[0 * CHUNK + row, pl.ds(k * L, L)] for k in range(D // L)]
                ip = [rows_v[1 * CHUNK + row, pl.ds(k * L, L)] for k in range(D // L)]
                inn = [rows_v[2 * CHUNK + row, pl.ds(k * L, L)] for k in range(D // L)]
                eip = [rows_v[3 * CHUNK + row, pl.ds(k * L, L)] for k in range(D // L)]
                ein = [rows_v[4 * CHUNK + row, pl.ds(k * L, L)] for k in range(D // L)]
                eup = [rows_v[5 * CHUNK + row, pl.ds(k * L, L)] for k in range(D // L)]
                eun = [rows_v[6 * CHUNK + row, pl.ds(k * L, L)] for k in range(D // L)]
                vcf = vi = vu = None
                for k in range(D // L):
                    tcf = u[k] * (ip[k] - inn[k])
                    a = ip[k] - eip[k]
                    b = ip[k] - ein[k]
                    ti = a * a - b * b
                    a = u[k] - eup[k]
                    b = u[k] - eun[k]
                    tu = a * a - b * b
                    vcf = tcf if vcf is None else vcf + tcf
                    vi = ti if vi is None else vi + ti
                    vu = tu if vu is None else vu + tu
                sidx = lanes16 + j
                plsc.store_scatter(tb[0], [sidx], vcf)
                plsc.store_scatter(tb[1], [sidx], vi)
                plsc.store_scatter(tb[2], [sidx], vu)
            zs = []
            for t in range(3):
                acc = tb[t][pl.ds(0, L)]
                for l in range(1, L):
                    acc = acc + tb[t][pl.ds(l * L, L)]
                zs.append(acc)
            zcf, zi, zu = zs
            off = c * CHUNK + p0
            mi = plsc.load_gather(pairs_v, [(lanes + off) * 9 + 5])
            mu = plsc.load_gather(pairs_v, [(lanes + off) * 9 + 8])
            out_v[0][pl.ds(off, L)] = zcf
            out_v[1][pl.ds(off, L)] = zi * mi.astype(jnp.float32)
            out_v[2][pl.ds(off, L)] = zu * mu.astype(jnp.float32)
            return carry2

        return lax.fori_loop(0, ngroup, group_body, carry)

    lax.fori_loop(0, nchunk, chunk_body, 0)
    for r in range(3):
        pltpu.sync_copy(out_v[r], out_hbm.at[pl.ds(r * B + base, pw)])


def _sc_pairs(x, pairs_flat):
    B = pairs_flat.shape[0] // 9
    mesh = plsc.VectorSubcoreMesh(
        core_axis_name="c", subcore_axis_name="s",
        num_cores=NC, num_subcores=NS)
    kfn = pl.kernel(
        _sc_body,
        out_type=jax.ShapeDtypeStruct((3 * B,), jnp.float32),
        mesh=mesh,
        compiler_params=pltpu.CompilerParams(
            needs_layout_passes=False, use_tc_tiling_on_sc=True),
        scratch_types=(
            [pltpu.VMEM((NG * CHUNK, D), jnp.float32)]
            + [pltpu.VMEM((B // NW,), jnp.float32)] * 3
            + [pltpu.VMEM((L * L,), jnp.float32)] * 3
            + [pltpu.VMEM((B // NW * 9,), jnp.int32)]
            + [pltpu.SemaphoreType.DMA]
        ),
    )
    return kfn(x, pairs_flat)


def _loss_body(z_ref, o_ref):
    z = z_ref[...]
    ls = jnp.minimum(z, 0.0) - jnp.log1p(jnp.exp(-jnp.abs(z)))
    total = -(jnp.sum(ls[0, :])
              + COFF * (jnp.sum(ls[1, :]) + jnp.sum(ls[2, :])))
    o_ref[...] = jnp.reshape(total, (1, 1))


def kernel(x, pos_neg_pair_t):
    B = pos_neg_pair_t.shape[0]
    pairs_flat = pos_neg_pair_t.astype(jnp.int32).reshape(-1)
    z = _sc_pairs(x, pairs_flat).reshape(3, B)
    loss2d = pl.pallas_call(
        _loss_body,
        out_shape=jax.ShapeDtypeStruct((1, 1), jnp.float32),
    )(z)
    return loss2d[0, 0]
